# baseline (speedup 1.0000x reference)
import functools

import jax
import jax.numpy as jnp
from jax import lax
from jax.experimental import pallas as pl
from jax.experimental.pallas import tpu as pltpu
from jax.experimental.pallas import tpu_sc as plsc

N = 10000
E = 320000
H = 128
L = 3
G = 64
C = 1

NC = 2
NS = 16
K = 128
EPT = E // NS
EPTP = 20480
CH = EPTP // K
IG = 8
NG = CH // IG
NBUF = 2
NA = N + 8
RPT = 640
RPT_LAST = N - (NS - 1) * RPT


def _seg_sum_z(h, src2d, dst2d):
    mesh = plsc.VectorSubcoreMesh(core_axis_name="c", subcore_axis_name="s",
                                  num_cores=NC, num_subcores=NS)

    @functools.partial(
        pl.kernel,
        out_type=jax.ShapeDtypeStruct((2 * N, H), jnp.float32),
        mesh=mesh,
        scratch_types=[
            pltpu.VMEM_SHARED((NA, H), jnp.float32),
            [pltpu.VMEM((IG, K), jnp.int32) for _ in range(2)],
            [pltpu.VMEM((IG, K), jnp.int32) for _ in range(2)],
            [pltpu.VMEM((K, H), jnp.float32) for _ in range(NBUF)],
            pltpu.SemaphoreType.DMA((NBUF,)),
            pltpu.SemaphoreType.DMA((NBUF,)),
            pltpu.SemaphoreType.DMA((2,)),
            pltpu.SemaphoreType.DMA((2,)),
        ],
    )
    def seg_kernel(h_hbm, src_hbm, dst_hbm, out_hbm, acc, srcv, dstv, rows,
                   gsem, scsem, ssem, dsem):
        c = lax.axis_index("c")
        s = lax.axis_index("s")
        w = c * NS + s
        r0 = s * RPT

        @pl.when(s < NS - 1)
        def _():
            pltpu.sync_copy(h_hbm.at[pl.ds(c * N + r0, RPT)],
                            acc.at[pl.ds(r0, RPT)])

        @pl.when(s == NS - 1)
        def _():
            pltpu.sync_copy(h_hbm.at[pl.ds(c * N + (NS - 1) * RPT, RPT_LAST)],
                            acc.at[pl.ds((NS - 1) * RPT, RPT_LAST)])

        pltpu.sync_copy(src_hbm.at[w, pl.ds(0, IG), :], srcv[0])
        pltpu.sync_copy(dst_hbm.at[w, pl.ds(0, IG), :], dstv[0])
        plsc.subcore_barrier()

        for b in range(NBUF):
            pltpu.async_copy(h_hbm.at[srcv[0].at[b]], rows[b], gsem.at[b])

        def idx_copies(g, buf):
            return (
                pltpu.make_async_copy(src_hbm.at[w, pl.ds(g * IG, IG), :],
                                      srcv[buf], ssem.at[buf]),
                pltpu.make_async_copy(dst_hbm.at[w, pl.ds(g * IG, IG), :],
                                      dstv[buf], dsem.at[buf]),
            )

        def fire_gather(sv, row, b):
            pltpu.async_copy(h_hbm.at[sv.at[row]], rows[b], gsem.at[b])

        def wait_gather(sv, row, b):
            pltpu.make_async_copy(h_hbm.at[sv.at[row]], rows[b],
                                  gsem.at[b]).wait()

        def fire_scatter(dv, row, b):
            pltpu.async_copy(rows[b], acc.at[dv.at[row]], scsem.at[b],
                             add=True)

        def wait_scatter(dv, row, b):
            pltpu.make_async_copy(rows[b], acc.at[dv.at[row]],
                                  scsem.at[b]).wait()

        @pl.loop(0, NG, step=2)
        def _(t):
            for parity in range(2):
                g = t + parity
                cur = parity
                nxt = 1 - parity
                has_next = g + 1 < NG

                @pl.when(has_next)
                def _():
                    for cp in idx_copies(g + 1, nxt):
                        cp.start()

                for jj in range(IG):
                    b = jj % NBUF
                    wait_gather(srcv[cur], jj, b)
                    fire_scatter(dstv[cur], jj, b)
                    pb = (jj - 1) % NBUF
                    rf = jj - 1 + NBUF
                    if jj == 0:
                        @pl.when(g > 0)
                        def _():
                            wait_scatter(dstv[nxt], IG - 1, pb)
                            fire_gather(srcv[cur], rf, pb)
                    elif rf < IG:
                        wait_scatter(dstv[cur], jj - 1, pb)
                        fire_gather(srcv[cur], rf, pb)
                    else:
                        @pl.when(has_next)
                        def _():
                            if rf == IG:
                                for cp in idx_copies(g + 1, nxt):
                                    cp.wait()
                            wait_scatter(dstv[cur], jj - 1, pb)
                            fire_gather(srcv[nxt], rf - IG, pb)

        for jj in range(IG - NBUF, IG):
            wait_scatter(dstv[(NG - 1) % 2], jj, jj % NBUF)
        plsc.subcore_barrier()

        @pl.when(s < NS - 1)
        def _():
            pltpu.sync_copy(acc.at[pl.ds(r0, RPT)],
                            out_hbm.at[pl.ds(c * N + r0, RPT)])

        @pl.when(s == NS - 1)
        def _():
            pltpu.sync_copy(acc.at[pl.ds((NS - 1) * RPT, RPT_LAST)],
                            out_hbm.at[pl.ds(c * N + (NS - 1) * RPT, RPT_LAST)])

    return seg_kernel(h, src2d, dst2d)


def _gin_mlp(z, W1, b1, W2, b2):
    BR = 1000
    NB = N // BR

    def body(z_ref, w1_ref, b1_ref, w2_ref, b2_ref, o_ref):
        zz = z_ref[...]
        t = jnp.maximum(
            jnp.dot(zz, w1_ref[0], preferred_element_type=jnp.float32) + b1_ref[0],
            0.0)
        o_ref[...] = jnp.maximum(
            jnp.dot(t, w2_ref[0], preferred_element_type=jnp.float32) + b2_ref[0],
            0.0)

    return pl.pallas_call(
        body,
        grid=(2, NB),
        in_specs=[
            pl.BlockSpec((BR, H), lambda e, i: (e * NB + i, 0)),
            pl.BlockSpec((1, H, H), lambda e, i: (e, 0, 0)),
            pl.BlockSpec((1, 1, H), lambda e, i: (e, 0, 0)),
            pl.BlockSpec((1, H, H), lambda e, i: (e, 0, 0)),
            pl.BlockSpec((1, 1, H), lambda e, i: (e, 0, 0)),
        ],
        out_specs=pl.BlockSpec((BR, H), lambda e, i: (e * NB + i, 0)),
        out_shape=jax.ShapeDtypeStruct((2 * N, H), jnp.float32),
    )(z, W1, b1, W2, b2)


def _head(h, batch1, batch2, W11, b11, W12, b12, W2a, W2b, b21, W22, b22):
    BR = 1000
    NB = N // BR

    def body(h1_ref, h2_ref, bt1_ref, bt2_ref, w11_ref, b11_ref, w12_ref,
             b12_ref, w2a_ref, w2b_ref, b21_ref, w22_ref, b22_ref, o_ref,
             s1, s2, c1, c2):
        i = pl.program_id(0)

        @pl.when(i == 0)
        def _():
            s1[...] = jnp.zeros_like(s1)
            s2[...] = jnp.zeros_like(s2)
            c1[...] = jnp.zeros_like(c1)
            c2[...] = jnp.zeros_like(c2)

        gids = lax.broadcasted_iota(jnp.int32, (G, BR), 0)
        oh1 = (bt1_ref[0] == gids).astype(jnp.float32)
        oh2 = (bt2_ref[0] == gids).astype(jnp.float32)
        s1[...] += jnp.dot(oh1, h1_ref[...], preferred_element_type=jnp.float32)
        s2[...] += jnp.dot(oh2, h2_ref[...], preferred_element_type=jnp.float32)
        c1[...] += jnp.sum(oh1, axis=1, keepdims=True)
        c2[...] += jnp.sum(oh2, axis=1, keepdims=True)

        @pl.when(i == NB - 1)
        def _():
            m1 = s1[...] / jnp.maximum(c1[...], 1.0)
            m2 = s2[...] / jnp.maximum(c2[...], 1.0)
            g1 = jnp.dot(m1, w11_ref[...], preferred_element_type=jnp.float32) + b11_ref[...]
            g2 = jnp.dot(m2, w12_ref[...], preferred_element_type=jnp.float32) + b12_ref[...]
            hg = jnp.maximum(
                jnp.dot(g1, w2a_ref[...], preferred_element_type=jnp.float32)
                + jnp.dot(g2, w2b_ref[...], preferred_element_type=jnp.float32)
                + b21_ref[...], 0.0)
            o_ref[...] = (jnp.dot(hg, w22_ref[...], preferred_element_type=jnp.float32)
                          + b22_ref[...])

    full = lambda shape: pl.BlockSpec(shape, lambda i: tuple(0 for _ in shape))
    return pl.pallas_call(
        body,
        grid=(NB,),
        in_specs=[
            pl.BlockSpec((BR, H), lambda i: (i, 0)),
            pl.BlockSpec((BR, H), lambda i: (i, 0)),
            pl.BlockSpec((1, 1, BR), lambda i: (i, 0, 0)),
            pl.BlockSpec((1, 1, BR), lambda i: (i, 0, 0)),
            full((H, H)), full((1, H)),
            full((H, H)), full((1, H)),
            full((H, H)), full((H, H)), full((1, H)),
            full((H, C)), full((1, C)),
        ],
        out_specs=full((G, C)),
        out_shape=jax.ShapeDtypeStruct((G, C), jnp.float32),
        scratch_shapes=[
            pltpu.VMEM((G, H), jnp.float32),
            pltpu.VMEM((G, H), jnp.float32),
            pltpu.VMEM((G, 1), jnp.float32),
            pltpu.VMEM((G, 1), jnp.float32),
        ],
    )(h[:N], h[N:], batch1.reshape(NB, 1, BR), batch2.reshape(NB, 1, BR),
      W11, b11, W12, b12, W2a, W2b, b21, W22, b22)


def kernel(x1, x2, edge_index1, edge_index2, batch1, batch2,
           gnn1_W1, gnn1_b1, gnn1_W2, gnn1_b2,
           gnn2_W1, gnn2_b1, gnn2_W2, gnn2_b2,
           mlp11_W, mlp11_b, mlp12_W, mlp12_b,
           mlp2_W1, mlp2_b1, mlp2_W2, mlp2_b2):
    h = jnp.concatenate([x1, x2], axis=0)
    pad = EPTP - EPT

    def _lay(a, off, sink):
        a = a.reshape(NS, EPT)
        fill = jnp.full((NS, pad), sink, dtype=jnp.int32) if sink >= 0 \
            else jnp.zeros((NS, pad), dtype=jnp.int32) + off
        return jnp.concatenate([a + (off if sink < 0 else 0), fill], axis=1)

    srcA = _lay(edge_index1[0], 0, -1)
    srcB = _lay(edge_index2[0], N, -1)
    dstA = _lay(edge_index1[1], 0, N)
    dstB = _lay(edge_index2[1], 0, N)
    src = jnp.concatenate([srcA, srcB]).reshape(NC * NS, CH, K)
    dst = jnp.concatenate([dstA, dstB]).reshape(NC * NS, CH, K)

    W1s = jnp.stack([gnn1_W1, gnn2_W1])
    W2s = jnp.stack([gnn1_W2, gnn2_W2])
    b1s = jnp.stack([gnn1_b1, gnn2_b1])[:, :, None, :]
    b2s = jnp.stack([gnn1_b2, gnn2_b2])[:, :, None, :]

    for l in range(L):
        z = _seg_sum_z(h, src, dst)
        h = _gin_mlp(z, W1s[:, l], b1s[:, l], W2s[:, l], b2s[:, l])

    return _head(h, batch1, batch2,
                 mlp11_W, mlp11_b[None, :], mlp12_W, mlp12_b[None, :],
                 mlp2_W1[:H], mlp2_W1[H:], mlp2_b1[None, :],
                 mlp2_W2, mlp2_b2[None, :])


# speedup vs baseline: 1.0388x; 1.0388x over previous
import functools

import jax
import jax.numpy as jnp
from jax import lax
from jax.experimental import pallas as pl
from jax.experimental.pallas import tpu as pltpu
from jax.experimental.pallas import tpu_sc as plsc

N = 10000
E = 320000
H = 128
L = 3
G = 64
C = 1

NC = 2
NS = 16
K = 80
EPT = E // NS
EPTP = 20480
CH = EPTP // K
IG = 16
NG = CH // IG
NBUF = 4
SLAG = 2
NA = N + 8
RPT = 640
RPT_LAST = N - (NS - 1) * RPT


def _seg_sum_z(h, src2d, dst2d):
    mesh = plsc.VectorSubcoreMesh(core_axis_name="c", subcore_axis_name="s",
                                  num_cores=NC, num_subcores=NS)

    @functools.partial(
        pl.kernel,
        out_type=jax.ShapeDtypeStruct((2 * N, H), jnp.float32),
        mesh=mesh,
        scratch_types=[
            pltpu.VMEM_SHARED((NA, H), jnp.float32),
            [pltpu.VMEM((IG, K), jnp.int32) for _ in range(2)],
            [pltpu.VMEM((IG, K), jnp.int32) for _ in range(2)],
            [pltpu.VMEM((K, H), jnp.float32) for _ in range(NBUF)],
            pltpu.SemaphoreType.DMA((NBUF,)),
            pltpu.SemaphoreType.DMA((NBUF,)),
            pltpu.SemaphoreType.DMA((2,)),
            pltpu.SemaphoreType.DMA((2,)),
        ],
    )
    def seg_kernel(h_hbm, src_hbm, dst_hbm, out_hbm, acc, srcv, dstv, rows,
                   gsem, scsem, ssem, dsem):
        c = lax.axis_index("c")
        s = lax.axis_index("s")
        w = c * NS + s
        r0 = s * RPT

        @pl.when(s < NS - 1)
        def _():
            pltpu.sync_copy(h_hbm.at[pl.ds(c * N + r0, RPT)],
                            acc.at[pl.ds(r0, RPT)])

        @pl.when(s == NS - 1)
        def _():
            pltpu.sync_copy(h_hbm.at[pl.ds(c * N + (NS - 1) * RPT, RPT_LAST)],
                            acc.at[pl.ds((NS - 1) * RPT, RPT_LAST)])

        pltpu.sync_copy(src_hbm.at[w, pl.ds(0, IG), :], srcv[0])
        pltpu.sync_copy(dst_hbm.at[w, pl.ds(0, IG), :], dstv[0])
        plsc.subcore_barrier()

        for b in range(NBUF):
            pltpu.async_copy(h_hbm.at[srcv[0].at[b]], rows[b], gsem.at[b])

        def idx_copies(g, buf):
            return (
                pltpu.make_async_copy(src_hbm.at[w, pl.ds(g * IG, IG), :],
                                      srcv[buf], ssem.at[buf]),
                pltpu.make_async_copy(dst_hbm.at[w, pl.ds(g * IG, IG), :],
                                      dstv[buf], dsem.at[buf]),
            )

        def fire_gather(sv, row, b):
            pltpu.async_copy(h_hbm.at[sv.at[row]], rows[b], gsem.at[b])

        def wait_gather(sv, row, b):
            pltpu.make_async_copy(h_hbm.at[sv.at[row]], rows[b],
                                  gsem.at[b]).wait()

        def fire_scatter(dv, row, b):
            pltpu.async_copy(rows[b], acc.at[dv.at[row]], scsem.at[b],
                             add=True)

        def wait_scatter(dv, row, b):
            pltpu.make_async_copy(rows[b], acc.at[dv.at[row]],
                                  scsem.at[b]).wait()

        @pl.loop(0, NG, step=2)
        def _(t):
            for parity in range(2):
                g = t + parity
                cur = parity
                nxt = 1 - parity
                has_next = g + 1 < NG

                @pl.when(has_next)
                def _():
                    for cp in idx_copies(g + 1, nxt):
                        cp.start()

                for jj in range(IG):
                    b = jj % NBUF
                    wait_gather(srcv[cur], jj, b)
                    fire_scatter(dstv[cur], jj, b)
                    pb = (jj - SLAG) % NBUF
                    rf = jj - SLAG + NBUF
                    if jj < SLAG:
                        @pl.when(g > 0)
                        def _():
                            wait_scatter(dstv[nxt], IG - SLAG + jj, pb)
                            fire_gather(srcv[cur], rf, pb)
                    elif rf < IG:
                        wait_scatter(dstv[cur], jj - SLAG, pb)
                        fire_gather(srcv[cur], rf, pb)
                    else:
                        @pl.when(has_next)
                        def _():
                            if rf == IG:
                                for cp in idx_copies(g + 1, nxt):
                                    cp.wait()
                            wait_scatter(dstv[cur], jj - SLAG, pb)
                            fire_gather(srcv[nxt], rf - IG, pb)

        for jj in range(IG - NBUF, IG):
            wait_scatter(dstv[(NG - 1) % 2], jj, jj % NBUF)
        plsc.subcore_barrier()

        @pl.when(s < NS - 1)
        def _():
            pltpu.sync_copy(acc.at[pl.ds(r0, RPT)],
                            out_hbm.at[pl.ds(c * N + r0, RPT)])

        @pl.when(s == NS - 1)
        def _():
            pltpu.sync_copy(acc.at[pl.ds((NS - 1) * RPT, RPT_LAST)],
                            out_hbm.at[pl.ds(c * N + (NS - 1) * RPT, RPT_LAST)])

    return seg_kernel(h, src2d, dst2d)


def _gin_mlp(z, W1, b1, W2, b2):
    BR = 1000
    NB = N // BR

    def body(z_ref, w1_ref, b1_ref, w2_ref, b2_ref, o_ref):
        zz = z_ref[...]
        t = jnp.maximum(
            jnp.dot(zz, w1_ref[0], preferred_element_type=jnp.float32) + b1_ref[0],
            0.0)
        o_ref[...] = jnp.maximum(
            jnp.dot(t, w2_ref[0], preferred_element_type=jnp.float32) + b2_ref[0],
            0.0)

    return pl.pallas_call(
        body,
        grid=(2, NB),
        in_specs=[
            pl.BlockSpec((BR, H), lambda e, i: (e * NB + i, 0)),
            pl.BlockSpec((1, H, H), lambda e, i: (e, 0, 0)),
            pl.BlockSpec((1, 1, H), lambda e, i: (e, 0, 0)),
            pl.BlockSpec((1, H, H), lambda e, i: (e, 0, 0)),
            pl.BlockSpec((1, 1, H), lambda e, i: (e, 0, 0)),
        ],
        out_specs=pl.BlockSpec((BR, H), lambda e, i: (e * NB + i, 0)),
        out_shape=jax.ShapeDtypeStruct((2 * N, H), jnp.float32),
    )(z, W1, b1, W2, b2)


def _head(h, batch1, batch2, W11, b11, W12, b12, W2a, W2b, b21, W22, b22):
    BR = 1000
    NB = N // BR

    def body(h1_ref, h2_ref, bt1_ref, bt2_ref, w11_ref, b11_ref, w12_ref,
             b12_ref, w2a_ref, w2b_ref, b21_ref, w22_ref, b22_ref, o_ref,
             s1, s2, c1, c2):
        i = pl.program_id(0)

        @pl.when(i == 0)
        def _():
            s1[...] = jnp.zeros_like(s1)
            s2[...] = jnp.zeros_like(s2)
            c1[...] = jnp.zeros_like(c1)
            c2[...] = jnp.zeros_like(c2)

        gids = lax.broadcasted_iota(jnp.int32, (G, BR), 0)
        oh1 = (bt1_ref[0] == gids).astype(jnp.float32)
        oh2 = (bt2_ref[0] == gids).astype(jnp.float32)
        s1[...] += jnp.dot(oh1, h1_ref[...], preferred_element_type=jnp.float32)
        s2[...] += jnp.dot(oh2, h2_ref[...], preferred_element_type=jnp.float32)
        c1[...] += jnp.sum(oh1, axis=1, keepdims=True)
        c2[...] += jnp.sum(oh2, axis=1, keepdims=True)

        @pl.when(i == NB - 1)
        def _():
            m1 = s1[...] / jnp.maximum(c1[...], 1.0)
            m2 = s2[...] / jnp.maximum(c2[...], 1.0)
            g1 = jnp.dot(m1, w11_ref[...], preferred_element_type=jnp.float32) + b11_ref[...]
            g2 = jnp.dot(m2, w12_ref[...], preferred_element_type=jnp.float32) + b12_ref[...]
            hg = jnp.maximum(
                jnp.dot(g1, w2a_ref[...], preferred_element_type=jnp.float32)
                + jnp.dot(g2, w2b_ref[...], preferred_element_type=jnp.float32)
                + b21_ref[...], 0.0)
            o_ref[...] = (jnp.dot(hg, w22_ref[...], preferred_element_type=jnp.float32)
                          + b22_ref[...])

    full = lambda shape: pl.BlockSpec(shape, lambda i: tuple(0 for _ in shape))
    return pl.pallas_call(
        body,
        grid=(NB,),
        in_specs=[
            pl.BlockSpec((BR, H), lambda i: (i, 0)),
            pl.BlockSpec((BR, H), lambda i: (i, 0)),
            pl.BlockSpec((1, 1, BR), lambda i: (i, 0, 0)),
            pl.BlockSpec((1, 1, BR), lambda i: (i, 0, 0)),
            full((H, H)), full((1, H)),
            full((H, H)), full((1, H)),
            full((H, H)), full((H, H)), full((1, H)),
            full((H, C)), full((1, C)),
        ],
        out_specs=full((G, C)),
        out_shape=jax.ShapeDtypeStruct((G, C), jnp.float32),
        scratch_shapes=[
            pltpu.VMEM((G, H), jnp.float32),
            pltpu.VMEM((G, H), jnp.float32),
            pltpu.VMEM((G, 1), jnp.float32),
            pltpu.VMEM((G, 1), jnp.float32),
        ],
    )(h[:N], h[N:], batch1.reshape(NB, 1, BR), batch2.reshape(NB, 1, BR),
      W11, b11, W12, b12, W2a, W2b, b21, W22, b22)


def kernel(x1, x2, edge_index1, edge_index2, batch1, batch2,
           gnn1_W1, gnn1_b1, gnn1_W2, gnn1_b2,
           gnn2_W1, gnn2_b1, gnn2_W2, gnn2_b2,
           mlp11_W, mlp11_b, mlp12_W, mlp12_b,
           mlp2_W1, mlp2_b1, mlp2_W2, mlp2_b2):
    h = jnp.concatenate([x1, x2], axis=0)
    pad = EPTP - EPT

    def _lay(a, off, sink):
        a = a.reshape(NS, EPT)
        fill = jnp.full((NS, pad), sink, dtype=jnp.int32) if sink >= 0 \
            else jnp.zeros((NS, pad), dtype=jnp.int32) + off
        return jnp.concatenate([a + (off if sink < 0 else 0), fill], axis=1)

    srcA = _lay(edge_index1[0], 0, -1)
    srcB = _lay(edge_index2[0], N, -1)
    dstA = _lay(edge_index1[1], 0, N)
    dstB = _lay(edge_index2[1], 0, N)
    src = jnp.concatenate([srcA, srcB]).reshape(NC * NS, CH, K)
    dst = jnp.concatenate([dstA, dstB]).reshape(NC * NS, CH, K)

    W1s = jnp.stack([gnn1_W1, gnn2_W1])
    W2s = jnp.stack([gnn1_W2, gnn2_W2])
    b1s = jnp.stack([gnn1_b1, gnn2_b1])[:, :, None, :]
    b2s = jnp.stack([gnn1_b2, gnn2_b2])[:, :, None, :]

    for l in range(L):
        z = _seg_sum_z(h, src, dst)
        h = _gin_mlp(z, W1s[:, l], b1s[:, l], W2s[:, l], b2s[:, l])

    return _head(h, batch1, batch2,
                 mlp11_W, mlp11_b[None, :], mlp12_W, mlp12_b[None, :],
                 mlp2_W1[:H], mlp2_W1[H:], mlp2_b1[None, :],
                 mlp2_W2, mlp2_b2[None, :])


# speedup vs baseline: 1.1410x; 1.0984x over previous
import functools

import jax
import jax.numpy as jnp
from jax import lax
from jax.experimental import pallas as pl
from jax.experimental.pallas import tpu as pltpu
from jax.experimental.pallas import tpu_sc as plsc

N = 10000
E = 320000
H = 128
L = 3
G = 64
C = 1

NC = 2
NS = 16
K = 80
EPT = E // NS
EPTP = 20480
CH = EPTP // K
IG = 16
NG = CH // IG
NBUF = 4
SLAG = 2
NA = N + 8
RPT = 640
RPT_LAST = N - (NS - 1) * RPT


def _seg_sum_z(h, src2d, dst2d):
    mesh = plsc.VectorSubcoreMesh(core_axis_name="c", subcore_axis_name="s",
                                  num_cores=NC, num_subcores=NS)

    @functools.partial(
        pl.kernel,
        out_type=jax.ShapeDtypeStruct((2 * N, H), jnp.float32),
        mesh=mesh,
        scratch_types=[
            pltpu.VMEM_SHARED((NA, H), jnp.float32),
            [pltpu.VMEM((IG, K), jnp.int32) for _ in range(2)],
            [pltpu.VMEM((IG, K), jnp.int32) for _ in range(2)],
            [pltpu.VMEM((K, H), jnp.float32) for _ in range(NBUF)],
            pltpu.SemaphoreType.DMA((NBUF,)),
            pltpu.SemaphoreType.DMA((NBUF,)),
            pltpu.SemaphoreType.DMA((2,)),
            pltpu.SemaphoreType.DMA((2,)),
        ],
    )
    def seg_kernel(h_hbm, src_hbm, dst_hbm, out_hbm, acc, srcv, dstv, rows,
                   gsem, scsem, ssem, dsem):
        c = lax.axis_index("c")
        s = lax.axis_index("s")
        w = c * NS + s
        r0 = s * RPT

        @pl.when(s < NS - 1)
        def _():
            pltpu.sync_copy(h_hbm.at[pl.ds(c * N + r0, RPT)],
                            acc.at[pl.ds(r0, RPT)])

        @pl.when(s == NS - 1)
        def _():
            pltpu.sync_copy(h_hbm.at[pl.ds(c * N + (NS - 1) * RPT, RPT_LAST)],
                            acc.at[pl.ds((NS - 1) * RPT, RPT_LAST)])

        pltpu.sync_copy(src_hbm.at[w, pl.ds(0, IG), :], srcv[0])
        pltpu.sync_copy(dst_hbm.at[w, pl.ds(0, IG), :], dstv[0])
        plsc.subcore_barrier()

        for b in range(NBUF):
            pltpu.async_copy(h_hbm.at[srcv[0].at[b]], rows[b], gsem.at[b])

        def idx_copies(g, buf):
            return (
                pltpu.make_async_copy(src_hbm.at[w, pl.ds(g * IG, IG), :],
                                      srcv[buf], ssem.at[buf]),
                pltpu.make_async_copy(dst_hbm.at[w, pl.ds(g * IG, IG), :],
                                      dstv[buf], dsem.at[buf]),
            )

        def fire_gather(sv, row, b):
            pltpu.async_copy(h_hbm.at[sv.at[row]], rows[b], gsem.at[b])

        def wait_gather(sv, row, b):
            pltpu.make_async_copy(h_hbm.at[sv.at[row]], rows[b],
                                  gsem.at[b]).wait()

        def fire_scatter(dv, row, b):
            pltpu.async_copy(rows[b], acc.at[dv.at[row]], scsem.at[b],
                             add=True)

        def wait_scatter(dv, row, b):
            pltpu.make_async_copy(rows[b], acc.at[dv.at[row]],
                                  scsem.at[b]).wait()

        @pl.loop(0, NG, step=2)
        def _(t):
            for parity in range(2):
                g = t + parity
                cur = parity
                nxt = 1 - parity
                has_next = g + 1 < NG

                @pl.when(has_next)
                def _():
                    for cp in idx_copies(g + 1, nxt):
                        cp.start()

                for jj in range(IG):
                    b = jj % NBUF
                    wait_gather(srcv[cur], jj, b)
                    pltpu.sync_copy(rows[b], acc.at[dstv[cur].at[jj]],
                                    add=True)
                    rf = jj + NBUF
                    if rf < IG:
                        fire_gather(srcv[cur], rf, b)
                    else:
                        @pl.when(has_next)
                        def _():
                            if rf == IG:
                                for cp in idx_copies(g + 1, nxt):
                                    cp.wait()
                            fire_gather(srcv[nxt], rf - IG, b)

        plsc.subcore_barrier()

        @pl.when(s < NS - 1)
        def _():
            pltpu.sync_copy(acc.at[pl.ds(r0, RPT)],
                            out_hbm.at[pl.ds(c * N + r0, RPT)])

        @pl.when(s == NS - 1)
        def _():
            pltpu.sync_copy(acc.at[pl.ds((NS - 1) * RPT, RPT_LAST)],
                            out_hbm.at[pl.ds(c * N + (NS - 1) * RPT, RPT_LAST)])

    return seg_kernel(h, src2d, dst2d)


def _gin_mlp(z, W1, b1, W2, b2):
    BR = 1000
    NB = N // BR

    def body(z_ref, w1_ref, b1_ref, w2_ref, b2_ref, o_ref):
        zz = z_ref[...]
        t = jnp.maximum(
            jnp.dot(zz, w1_ref[0], preferred_element_type=jnp.float32) + b1_ref[0],
            0.0)
        o_ref[...] = jnp.maximum(
            jnp.dot(t, w2_ref[0], preferred_element_type=jnp.float32) + b2_ref[0],
            0.0)

    return pl.pallas_call(
        body,
        grid=(2, NB),
        in_specs=[
            pl.BlockSpec((BR, H), lambda e, i: (e * NB + i, 0)),
            pl.BlockSpec((1, H, H), lambda e, i: (e, 0, 0)),
            pl.BlockSpec((1, 1, H), lambda e, i: (e, 0, 0)),
            pl.BlockSpec((1, H, H), lambda e, i: (e, 0, 0)),
            pl.BlockSpec((1, 1, H), lambda e, i: (e, 0, 0)),
        ],
        out_specs=pl.BlockSpec((BR, H), lambda e, i: (e * NB + i, 0)),
        out_shape=jax.ShapeDtypeStruct((2 * N, H), jnp.float32),
    )(z, W1, b1, W2, b2)


def _head(h, batch1, batch2, W11, b11, W12, b12, W2a, W2b, b21, W22, b22):
    BR = 1000
    NB = N // BR

    def body(h1_ref, h2_ref, bt1_ref, bt2_ref, w11_ref, b11_ref, w12_ref,
             b12_ref, w2a_ref, w2b_ref, b21_ref, w22_ref, b22_ref, o_ref,
             s1, s2, c1, c2):
        i = pl.program_id(0)

        @pl.when(i == 0)
        def _():
            s1[...] = jnp.zeros_like(s1)
            s2[...] = jnp.zeros_like(s2)
            c1[...] = jnp.zeros_like(c1)
            c2[...] = jnp.zeros_like(c2)

        gids = lax.broadcasted_iota(jnp.int32, (G, BR), 0)
        oh1 = (bt1_ref[0] == gids).astype(jnp.float32)
        oh2 = (bt2_ref[0] == gids).astype(jnp.float32)
        s1[...] += jnp.dot(oh1, h1_ref[...], preferred_element_type=jnp.float32)
        s2[...] += jnp.dot(oh2, h2_ref[...], preferred_element_type=jnp.float32)
        c1[...] += jnp.sum(oh1, axis=1, keepdims=True)
        c2[...] += jnp.sum(oh2, axis=1, keepdims=True)

        @pl.when(i == NB - 1)
        def _():
            m1 = s1[...] / jnp.maximum(c1[...], 1.0)
            m2 = s2[...] / jnp.maximum(c2[...], 1.0)
            g1 = jnp.dot(m1, w11_ref[...], preferred_element_type=jnp.float32) + b11_ref[...]
            g2 = jnp.dot(m2, w12_ref[...], preferred_element_type=jnp.float32) + b12_ref[...]
            hg = jnp.maximum(
                jnp.dot(g1, w2a_ref[...], preferred_element_type=jnp.float32)
                + jnp.dot(g2, w2b_ref[...], preferred_element_type=jnp.float32)
                + b21_ref[...], 0.0)
            o_ref[...] = (jnp.dot(hg, w22_ref[...], preferred_element_type=jnp.float32)
                          + b22_ref[...])

    full = lambda shape: pl.BlockSpec(shape, lambda i: tuple(0 for _ in shape))
    return pl.pallas_call(
        body,
        grid=(NB,),
        in_specs=[
            pl.BlockSpec((BR, H), lambda i: (i, 0)),
            pl.BlockSpec((BR, H), lambda i: (i, 0)),
            pl.BlockSpec((1, 1, BR), lambda i: (i, 0, 0)),
            pl.BlockSpec((1, 1, BR), lambda i: (i, 0, 0)),
            full((H, H)), full((1, H)),
            full((H, H)), full((1, H)),
            full((H, H)), full((H, H)), full((1, H)),
            full((H, C)), full((1, C)),
        ],
        out_specs=full((G, C)),
        out_shape=jax.ShapeDtypeStruct((G, C), jnp.float32),
        scratch_shapes=[
            pltpu.VMEM((G, H), jnp.float32),
            pltpu.VMEM((G, H), jnp.float32),
            pltpu.VMEM((G, 1), jnp.float32),
            pltpu.VMEM((G, 1), jnp.float32),
        ],
    )(h[:N], h[N:], batch1.reshape(NB, 1, BR), batch2.reshape(NB, 1, BR),
      W11, b11, W12, b12, W2a, W2b, b21, W22, b22)


def kernel(x1, x2, edge_index1, edge_index2, batch1, batch2,
           gnn1_W1, gnn1_b1, gnn1_W2, gnn1_b2,
           gnn2_W1, gnn2_b1, gnn2_W2, gnn2_b2,
           mlp11_W, mlp11_b, mlp12_W, mlp12_b,
           mlp2_W1, mlp2_b1, mlp2_W2, mlp2_b2):
    h = jnp.concatenate([x1, x2], axis=0)
    pad = EPTP - EPT

    def _lay(a, off, sink):
        a = a.reshape(NS, EPT)
        fill = jnp.full((NS, pad), sink, dtype=jnp.int32) if sink >= 0 \
            else jnp.zeros((NS, pad), dtype=jnp.int32) + off
        return jnp.concatenate([a + (off if sink < 0 else 0), fill], axis=1)

    srcA = _lay(edge_index1[0], 0, -1)
    srcB = _lay(edge_index2[0], N, -1)
    dstA = _lay(edge_index1[1], 0, N)
    dstB = _lay(edge_index2[1], 0, N)
    src = jnp.concatenate([srcA, srcB]).reshape(NC * NS, CH, K)
    dst = jnp.concatenate([dstA, dstB]).reshape(NC * NS, CH, K)

    W1s = jnp.stack([gnn1_W1, gnn2_W1])
    W2s = jnp.stack([gnn1_W2, gnn2_W2])
    b1s = jnp.stack([gnn1_b1, gnn2_b1])[:, :, None, :]
    b2s = jnp.stack([gnn1_b2, gnn2_b2])[:, :, None, :]

    for l in range(L):
        z = _seg_sum_z(h, src, dst)
        h = _gin_mlp(z, W1s[:, l], b1s[:, l], W2s[:, l], b2s[:, l])

    return _head(h, batch1, batch2,
                 mlp11_W, mlp11_b[None, :], mlp12_W, mlp12_b[None, :],
                 mlp2_W1[:H], mlp2_W1[H:], mlp2_b1[None, :],
                 mlp2_W2, mlp2_b2[None, :])
